# Initial kernel scaffold; baseline (speedup 1.0000x reference)
import jax
import jax.numpy as jnp
from jax.experimental import pallas as pl


def kernel(initial_position_indexes, destination_indexes, W_initial, W_destination):
    raise NotImplementedError("write your pallas kernel here")


# speedup vs baseline: 12.9775x; 12.9775x over previous
import jax
import jax.numpy as jnp
from jax import lax
from jax.experimental import pallas as pl
from jax.experimental.pallas import tpu as pltpu
from jax.experimental.pallas import tpu_sc as plsc

EMBED = 128
NSQ = 64
NCOMB = NSQ * NSQ
LANES = 16
CHUNK = 128


def _make_sc_lookup(bl):
    info = plsc.get_sparse_core_info()
    n_workers = info.num_cores * info.num_subcores
    tpw = bl // n_workers
    assert tpw % CHUNK == 0
    n_chunks = tpw // CHUNK
    rows_per_tile = NCOMB // 16

    mesh = plsc.VectorSubcoreMesh(core_axis_name="c", subcore_axis_name="s")

    def body(i1_hbm, i2_hbm, w1_hbm, w2_hbm, out_hbm,
             w1_v, w2_v, wtile_v, idx1_v, idx2_v, idxc_v, rows_v, sem_g, wsum):
        cid = lax.axis_index("c")
        sid = lax.axis_index("s")
        wid = sid * info.num_cores + cid

        if True:
            pltpu.sync_copy(w1_hbm, w1_v)
            pltpu.sync_copy(w2_hbm, w2_v)

            def build_row(r, carry):
                row = sid * rows_per_tile + r
                a = row // NSQ
                b = lax.rem(row, NSQ)
                for k in range(EMBED // LANES):
                    v = (w1_v[pl.ds(a * EMBED + k * LANES, LANES)]
                         + w2_v[pl.ds(b * EMBED + k * LANES, LANES)])
                    wtile_v[r, pl.ds(k * LANES, LANES)] = v
                return carry

            lax.fori_loop(0, rows_per_tile, build_row, 0)
            pltpu.sync_copy(
                wtile_v,
                wsum.at[pl.ds(sid * rows_per_tile, rows_per_tile)],
            )
            plsc.subcore_barrier()

            base0 = wid * tpw

            def step(g, carry):
                base = base0 + g * CHUNK
                pltpu.sync_copy(i1_hbm.at[pl.ds(base, CHUNK)], idx1_v)
                pltpu.sync_copy(i2_hbm.at[pl.ds(base, CHUNK)], idx2_v)
                for k in range(CHUNK // LANES):
                    s = pl.ds(k * LANES, LANES)
                    idxc_v[s] = idx1_v[s] * NSQ + idx2_v[s]
                pltpu.async_copy(wsum.at[idxc_v], rows_v, sem_g).wait()
                pltpu.sync_copy(rows_v, out_hbm.at[pl.ds(base, CHUNK)])
                return carry

            lax.fori_loop(0, n_chunks, step, 0)

    return pl.kernel(
        body,
        out_type=jax.ShapeDtypeStruct((bl, EMBED), jnp.float32),
        mesh=mesh,
        scratch_types=[
            pltpu.VMEM((NSQ * EMBED,), jnp.float32),
            pltpu.VMEM((NSQ * EMBED,), jnp.float32),
            pltpu.VMEM((rows_per_tile, EMBED), jnp.float32),
            pltpu.VMEM((CHUNK,), jnp.int32),
            pltpu.VMEM((CHUNK,), jnp.int32),
            pltpu.VMEM((CHUNK,), jnp.int32),
            pltpu.VMEM((CHUNK, EMBED), jnp.float32),
            pltpu.SemaphoreType.DMA,
            pltpu.VMEM_SHARED((NCOMB, EMBED), jnp.float32),
        ],
    )


def kernel(initial_position_indexes, destination_indexes, W_initial, W_destination):
    b, l = initial_position_indexes.shape
    bl = b * l
    i1 = initial_position_indexes.reshape(bl).astype(jnp.int32)
    i2 = destination_indexes.reshape(bl).astype(jnp.int32)
    w1 = W_initial.reshape(NSQ * EMBED)
    w2 = W_destination.reshape(NSQ * EMBED)
    out = _make_sc_lookup(bl)(i1, i2, w1, w2)
    return out.reshape(b, l, EMBED)


# speedup vs baseline: 23.9176x; 1.8430x over previous
import jax
import jax.numpy as jnp
from jax import lax
from jax.experimental import pallas as pl
from jax.experimental.pallas import tpu as pltpu
from jax.experimental.pallas import tpu_sc as plsc

EMBED = 128
NSQ = 64
NCOMB = NSQ * NSQ
LANES = 16
IDXW = 128
CHUNK = 256
NGATH = CHUNK // IDXW


def _make_sc_lookup(bl):
    info = plsc.get_sparse_core_info()
    n_workers = info.num_cores * info.num_subcores
    tpw = bl // n_workers
    assert tpw % CHUNK == 0
    n_chunks = tpw // CHUNK
    assert n_chunks % 2 == 0 and n_chunks >= 4
    rows_per_tile = NCOMB // 16

    mesh = plsc.VectorSubcoreMesh(core_axis_name="c", subcore_axis_name="s")

    def body(i1_hbm, i2_hbm, w1_hbm, w2_hbm, out_hbm,
             w1_v, w2_v, idx1_v, idx2_v, idxc_v, rows_v, sem_g, sem_s, wsum):
        cid = lax.axis_index("c")
        sid = lax.axis_index("s")
        wid = sid * info.num_cores + cid

        pltpu.sync_copy(w1_hbm, w1_v)
        pltpu.sync_copy(w2_hbm, w2_v)

        def build_row(r, carry):
            row = sid * rows_per_tile + r
            a = row // NSQ
            b = lax.rem(row, NSQ)
            for k in range(EMBED // LANES):
                v = (w1_v[pl.ds(a * EMBED + k * LANES, LANES)]
                     + w2_v[pl.ds(b * EMBED + k * LANES, LANES)])
                rows_v[0][r, pl.ds(k * LANES, LANES)] = v
            return carry

        lax.fori_loop(0, rows_per_tile, build_row, 0)
        pltpu.sync_copy(rows_v[0], wsum.at[pl.ds(sid * rows_per_tile, rows_per_tile)])
        plsc.subcore_barrier()

        base0 = wid * tpw

        def load_and_fire(g, buf):
            base = base0 + g * CHUNK
            pltpu.sync_copy(i1_hbm.at[pl.ds(base, CHUNK)], idx1_v[buf])
            pltpu.sync_copy(i2_hbm.at[pl.ds(base, CHUNK)], idx2_v[buf])
            for j in range(NGATH):
                for k in range(IDXW // LANES):
                    s = pl.ds(j * IDXW + k * LANES, LANES)
                    idxc_v[buf][j, pl.ds(k * LANES, LANES)] = (
                        idx1_v[buf][s] * NSQ + idx2_v[buf][s])
            for j in range(NGATH):
                pltpu.async_copy(
                    wsum.at[idxc_v[buf].at[j]],
                    rows_v[buf].at[pl.ds(j * IDXW, IDXW)],
                    sem_g[buf])

        def wait_gather(buf):
            pltpu.make_async_copy(
                wsum.at[idxc_v[buf].at[0]],
                rows_v[buf].at[pl.ds(0, IDXW)],
                sem_g[buf]).wait()

        def fire_store(g, buf):
            base = base0 + g * CHUNK
            pltpu.async_copy(rows_v[buf], out_hbm.at[pl.ds(base, CHUNK)],
                             sem_s[buf])

        def wait_store(buf):
            pltpu.make_async_copy(rows_v[buf], out_hbm.at[pl.ds(0, CHUNK)],
                                  sem_s[buf]).wait()

        def do_g(g, buf, first, last):
            nbuf = 1 - buf
            if not first:
                wait_store(nbuf)
            if not last:
                load_and_fire(g + 1, nbuf)
            for _ in range(NGATH):
                wait_gather(buf)
            fire_store(g, buf)

        load_and_fire(0, 0)
        do_g(0, 0, first=True, last=False)

        def step(p, carry):
            do_g(2 * p + 1, 1, first=False, last=False)
            do_g(2 * p + 2, 0, first=False, last=False)
            return carry

        lax.fori_loop(0, n_chunks // 2 - 1, step, 0)
        do_g(n_chunks - 1, 1, first=False, last=True)
        wait_store(1)

    return pl.kernel(
        body,
        out_type=jax.ShapeDtypeStruct((bl, EMBED), jnp.float32),
        mesh=mesh,
        scratch_types=[
            pltpu.VMEM((NSQ * EMBED,), jnp.float32),
            pltpu.VMEM((NSQ * EMBED,), jnp.float32),
            [pltpu.VMEM((CHUNK,), jnp.int32) for _ in range(2)],
            [pltpu.VMEM((CHUNK,), jnp.int32) for _ in range(2)],
            [pltpu.VMEM((NGATH, IDXW), jnp.int32) for _ in range(2)],
            [pltpu.VMEM((CHUNK, EMBED), jnp.float32) for _ in range(2)],
            [pltpu.SemaphoreType.DMA for _ in range(2)],
            [pltpu.SemaphoreType.DMA for _ in range(2)],
            pltpu.VMEM_SHARED((NCOMB, EMBED), jnp.float32),
        ],
    )


def kernel(initial_position_indexes, destination_indexes, W_initial, W_destination):
    b, l = initial_position_indexes.shape
    bl = b * l
    i1 = initial_position_indexes.reshape(bl).astype(jnp.int32)
    i2 = destination_indexes.reshape(bl).astype(jnp.int32)
    w1 = W_initial.reshape(NSQ * EMBED)
    w2 = W_destination.reshape(NSQ * EMBED)
    out = _make_sc_lookup(bl)(i1, i2, w1, w2)
    return out.reshape(b, l, EMBED)
